# baseline (speedup 1.0000x reference)
import functools

import jax
import jax.numpy as jnp
from jax import lax
from jax.experimental import pallas as pl
from jax.experimental.pallas import tpu as pltpu
from jax.experimental.pallas import tpu_sc as plsc

B = 16384
CTX = 10
NEG = 5
D = 64
NVJ = NEG + 1
LANES = 16
NCHUNK = D // LANES
NC = 2
NS = 16
NW = NC * NS
BPW = B // NW
CB = 64
NBLK = BPW // CB
CIDX_ROWS = CB * CTX // 128
VIDX_ROWS = CB * NVJ // 128


def _sc_partials(ctx2d, vidx2d, u_table, v_table):
    mesh = plsc.VectorSubcoreMesh(core_axis_name="c", subcore_axis_name="s")

    @functools.partial(
        pl.kernel,
        out_type=jax.ShapeDtypeStruct((B * NVJ, LANES), jnp.float32),
        mesh=mesh,
        compiler_params=pltpu.CompilerParams(use_tc_tiling_on_sc=False),
        scratch_types=[
            pltpu.VMEM((CB * CTX,), jnp.int32),
            pltpu.VMEM((CB * NVJ,), jnp.int32),
            pltpu.VMEM((CB * CTX, D), jnp.float32),
            pltpu.VMEM((CB * NVJ, D), jnp.float32),
            pltpu.VMEM((CB * NVJ, LANES), jnp.float32),
            pltpu.SemaphoreType.DMA,
        ],
    )
    def k(u_hbm, v_hbm, cidx_hbm, vidx_hbm, out_hbm,
          cidx_v, vidx_v, urows_v, vrows_v, part_v, sem):
        wid = lax.axis_index("s") * NC + lax.axis_index("c")

        @pl.loop(0, NBLK)
        def _block(nb):
            m = wid * NBLK + nb
            pltpu.sync_copy(cidx_hbm.at[pl.ds(m * CB * CTX, CB * CTX)],
                            cidx_v)
            pltpu.sync_copy(vidx_hbm.at[pl.ds(m * CB * NVJ, CB * NVJ)],
                            vidx_v)
            copies = []
            for r in range(CIDX_ROWS):
                copies.append(pltpu.async_copy(
                    u_hbm.at[cidx_v.at[pl.ds(r * 128, 128)]],
                    urows_v.at[pl.ds(r * 128, 128)], sem))
            for r in range(VIDX_ROWS):
                copies.append(pltpu.async_copy(
                    v_hbm.at[vidx_v.at[pl.ds(r * 128, 128)]],
                    vrows_v.at[pl.ds(r * 128, 128)], sem))
            for c in copies:
                c.wait()

            @pl.loop(0, CB)
            def _elem(i):
                u_chunks = []
                for c in range(NCHUNK):
                    sl = pl.ds(c * LANES, LANES)
                    acc = urows_v[i * CTX, sl]
                    for j in range(1, CTX):
                        acc = acc + urows_v[i * CTX + j, sl]
                    u_chunks.append(acc * (1.0 / CTX))
                for j in range(NVJ):
                    s = u_chunks[0] * vrows_v[i * NVJ + j, pl.ds(0, LANES)]
                    for c in range(1, NCHUNK):
                        s = s + u_chunks[c] * vrows_v[i * NVJ + j,
                                                      pl.ds(c * LANES, LANES)]
                    part_v[i * NVJ + j, :] = s if j == 0 else -s

            pltpu.sync_copy(part_v,
                            out_hbm.at[pl.ds(m * CB * NVJ, CB * NVJ)])

    return k(u_table, v_table, ctx2d, vidx2d)


def _tc_loss(partials):
    R = B * NVJ
    BLK = 8192
    grid = (R // BLK,)

    def body(p_ref, o_ref):
        i = pl.program_id(0)

        @pl.when(i == 0)
        def _():
            o_ref[...] = jnp.zeros_like(o_ref)

        s = jnp.sum(p_ref[...], axis=1)
        o_ref[...] += -jnp.sum(jax.nn.log_sigmoid(s))[None, None]

    out = pl.pallas_call(
        body,
        grid=grid,
        in_specs=[pl.BlockSpec((BLK, LANES), lambda i: (i, 0))],
        out_specs=pl.BlockSpec((1, 1), lambda i: (0, 0)),
        out_shape=jax.ShapeDtypeStruct((1, 1), jnp.float32),
    )(partials)
    return out[0, 0]


def kernel(context, target, negatives, u_table, v_table):
    ctx_flat = context.astype(jnp.int32).reshape(B * CTX)
    vidx_flat = jnp.concatenate(
        [target[:, None], negatives], axis=1).astype(jnp.int32).reshape(
            B * NVJ)
    partials = _sc_partials(ctx_flat, vidx_flat, u_table, v_table)
    return _tc_loss(partials)
